# baseline (speedup 1.0000x reference)
import jax
import jax.numpy as jnp
from jax.experimental import pallas as pl
from jax.experimental.pallas import tpu as pltpu

_DIM = 512
_N_BUCKETS = 1024
_BANDWIDTH = 4.0
_N_HASHES = 128


def _lsh_block_kernel(x_ref, rv_ref, out_ref):
    proj = jnp.dot(x_ref[...], rv_ref[...], preferred_element_type=jnp.float32)
    buckets = jnp.floor(proj * (1.0 / _BANDWIDTH)).astype(jnp.int32)
    out_ref[...] = (buckets & (_N_BUCKETS - 1)).astype(jnp.float32)


def kernel(x, random_vectors):
    n = x.shape[0]
    block = 6144
    return pl.pallas_call(
        _lsh_block_kernel,
        grid=(n // block,),
        in_specs=[
            pl.BlockSpec((block, _DIM), lambda i: (i, 0)),
            pl.BlockSpec((_DIM, _N_HASHES), lambda i: (0, 0)),
        ],
        out_specs=pl.BlockSpec((block, _N_HASHES), lambda i: (i, 0)),
        out_shape=jax.ShapeDtypeStruct((n, _N_HASHES), jnp.float32),
        compiler_params=pltpu.CompilerParams(
            dimension_semantics=("parallel",),
        ),
    )(x, random_vectors)
